# Initial kernel scaffold; baseline (speedup 1.0000x reference)
import jax
import jax.numpy as jnp
from jax.experimental import pallas as pl


def kernel(x, edge_index, W1, b1, W2, b2):
    raise NotImplementedError("write your pallas kernel here")


# speedup vs baseline: 2.6341x; 2.6341x over previous
import jax
import jax.numpy as jnp
from jax.experimental import pallas as pl

_N = 100000
_E = 1600000


def _mm_kernel(x_ref, w_ref, o_ref):
    o_ref[...] = jnp.dot(x_ref[...], w_ref[...], preferred_element_type=jnp.float32)


def _matmul(x, w, bn):
    n, k = x.shape
    m = w.shape[1]
    return pl.pallas_call(
        _mm_kernel,
        grid=(n // bn,),
        in_specs=[
            pl.BlockSpec((bn, k), lambda i: (i, 0)),
            pl.BlockSpec((k, m), lambda i: (0, 0)),
        ],
        out_specs=pl.BlockSpec((bn, m), lambda i: (i, 0)),
        out_shape=jax.ShapeDtypeStruct((n, m), jnp.float32),
    )(x, w)


def kernel(x, edge_index, W1, b1, W2, b2):
    src = edge_index[0]
    dst = edge_index[1]

    deg = jax.ops.segment_sum(jnp.ones((_E,), jnp.float32), dst, num_segments=_N) + 1.0
    dinv = jax.lax.rsqrt(deg)

    W1p = jnp.pad(W1, ((0, 0), (0, 64 - W1.shape[1])))
    h = _matmul(x, W1p, 2000)
    g = h * dinv[:, None]
    agg = jax.ops.segment_sum(g[src], dst, num_segments=_N) + g
    h1 = jax.nn.relu(agg * dinv[:, None] + jnp.pad(b1, (0, 14))[None, :])

    W2p = jnp.pad(W2, ((0, 14), (0, 0)))
    h2 = h1 @ W2p
    g2 = h2 * dinv[:, None]
    agg2 = jax.ops.segment_sum(g2[src], dst, num_segments=_N) + g2
    out = agg2 * dinv[:, None] + b2[None, :]
    return jax.nn.sigmoid(out)


# speedup vs baseline: 23.5564x; 8.9430x over previous
import functools

import jax
import jax.numpy as jnp
from jax import lax
from jax.experimental import pallas as pl
from jax.experimental.pallas import tpu as pltpu
from jax.experimental.pallas import tpu_sc as plsc

_N = 100000
_E = 1600000
_NP = 100096
_SLICE = _NP // 16
_B = 1000
_NBI = _SLICE // _B
_TAIL = _SLICE - _NBI * _B

_mesh = plsc.VectorSubcoreMesh(core_axis_name="c", subcore_axis_name="s")


def _init_acc2d(src_hbm, acc, rows_v, rows_t, base_r, fixed_block):
    def blk(k, _):
        r0 = 0 if fixed_block else base_r + k * _B
        pltpu.sync_copy(src_hbm.at[pl.ds(r0, _B), :], rows_v)
        pltpu.sync_copy(rows_v, acc.at[pl.ds(base_r + k * _B, _B), :])
        return _

    lax.fori_loop(0, _NBI, blk, None)
    r0 = 0 if fixed_block else base_r + _NBI * _B
    pltpu.sync_copy(src_hbm.at[pl.ds(r0, 256), :], rows_t)
    pltpu.sync_copy(rows_t, acc.at[pl.ds(base_r + _NBI * _B, 256), :])


def _writeout_acc2d(acc, out_hbm, rows_v, rows_t, base_r):
    def blk(k, _):
        pltpu.sync_copy(acc.at[pl.ds(base_r + k * _B, _B), :], rows_v)
        pltpu.sync_copy(rows_v, out_hbm.at[pl.ds(base_r + k * _B, _B), :])
        return _

    lax.fori_loop(0, _NBI, blk, None)
    pltpu.sync_copy(acc.at[pl.ds(base_r + _NBI * _B, 256), :], rows_t)
    pltpu.sync_copy(rows_t, out_hbm.at[pl.ds(base_r + _NBI * _B, 256), :])


def _deg_body(dst_h, ones_h, zeros_h, d0_h, d1_h,
              acc, idx_v, ones_v, buf_v, buf_t):
    c = lax.axis_index("c")
    s = lax.axis_index("s")
    base_r = s * _SLICE

    pltpu.sync_copy(ones_h, ones_v)

    def iblk(k, _):
        pltpu.sync_copy(zeros_h.at[pl.ds(0, _B)], buf_v)
        pltpu.sync_copy(buf_v, acc.at[pl.ds(base_r + k * _B, _B)])
        return _

    lax.fori_loop(0, _NBI, iblk, None)
    pltpu.sync_copy(zeros_h.at[pl.ds(0, 256)], buf_t)
    pltpu.sync_copy(buf_t, acc.at[pl.ds(base_r + _NBI * _B, 256)])
    plsc.subcore_barrier()

    ebase = (c * 16 + s) * (_E // 32)

    def eblk(k, _):
        pltpu.sync_copy(dst_h.at[pl.ds(ebase + k * _B, _B)], idx_v)
        pltpu.sync_copy(ones_v, acc.at[idx_v], add=True)
        return _

    lax.fori_loop(0, (_E // 32) // _B, eblk, None)
    plsc.subcore_barrier()

    def wout(out_h):
        def wblk(k, _):
            pltpu.sync_copy(acc.at[pl.ds(base_r + k * _B, _B)], buf_v)
            pltpu.sync_copy(buf_v, out_h.at[pl.ds(base_r + k * _B, _B)])
            return _

        lax.fori_loop(0, _NBI, wblk, None)
        pltpu.sync_copy(acc.at[pl.ds(base_r + _NBI * _B, 256)], buf_t)
        pltpu.sync_copy(buf_t, out_h.at[pl.ds(base_r + _NBI * _B, 256)])

    pl.when(c == 0)(lambda: wout(d0_h))
    pl.when(c == 1)(lambda: wout(d1_h))


def _make_deg_kernel():
    return pl.kernel(
        _deg_body,
        out_type=(
            jax.ShapeDtypeStruct((_NP,), jnp.float32),
            jax.ShapeDtypeStruct((_NP,), jnp.float32),
        ),
        mesh=_mesh,
        compiler_params=pltpu.CompilerParams(use_tc_tiling_on_sc=False),
        scratch_types=[
            pltpu.VMEM_SHARED((_NP,), jnp.float32),
            pltpu.VMEM((_B,), jnp.int32),
            pltpu.VMEM((_B,), jnp.float32),
            pltpu.VMEM((_B,), jnp.float32),
            pltpu.VMEM((256,), jnp.float32),
        ],
    )


def _prop_body(tab0_h, tab1_h, init1_h, src_h, dst_h, out0_h, out1_h,
               acc, idx_s, idx_d, rows_v, rows_t, sem,
               *, edges_per_core, init1_fixed):
    c = lax.axis_index("c")
    s = lax.axis_index("s")
    base_r = s * _SLICE
    nblk = edges_per_core // 16 // _B

    def run(tab, init_h, out_h, fixed, ebase0):
        _init_acc2d(init_h, acc, rows_v, rows_t, base_r, fixed)
        plsc.subcore_barrier()

        ebase = ebase0 + s * (edges_per_core // 16)

        def eblk(k, _):
            off = ebase + k * _B
            pltpu.sync_copy(src_h.at[pl.ds(off, _B)], idx_s)
            pltpu.async_copy(tab.at[idx_s], rows_v, sem).wait()
            pltpu.sync_copy(dst_h.at[pl.ds(off, _B)], idx_d)
            pltpu.sync_copy(rows_v, acc.at[idx_d], add=True)
            return _

        lax.fori_loop(0, nblk, eblk, None)
        plsc.subcore_barrier()
        _writeout_acc2d(acc, out_h, rows_v, rows_t, base_r)

    if init1_fixed:
        pl.when(c == 0)(lambda: run(tab0_h, tab0_h, out0_h, False, 0))
        pl.when(c == 1)(lambda: run(tab1_h, init1_h, out1_h, True,
                                    edges_per_core))
    else:
        pl.when(c == 0)(lambda: run(tab0_h, tab0_h, out0_h, False, 0))
        pl.when(c == 1)(lambda: run(tab1_h, tab1_h, out1_h, False, 0))


def _make_prop(edges_per_core, init1_fixed):
    return pl.kernel(
        functools.partial(_prop_body, edges_per_core=edges_per_core,
                          init1_fixed=init1_fixed),
        out_type=(
            jax.ShapeDtypeStruct((_NP, 16), jnp.float32),
            jax.ShapeDtypeStruct((_NP, 16), jnp.float32),
        ),
        mesh=_mesh,
        compiler_params=pltpu.CompilerParams(use_tc_tiling_on_sc=False),
        scratch_types=[
            pltpu.VMEM_SHARED((_NP, 16), jnp.float32),
            pltpu.VMEM((_B,), jnp.int32),
            pltpu.VMEM((_B,), jnp.int32),
            pltpu.VMEM((_B, 16), jnp.float32),
            pltpu.VMEM((256, 16), jnp.float32),
            pltpu.SemaphoreType.DMA,
        ],
    )


def _dinv_kernel(d0_ref, d1_ref, o_ref):
    o_ref[...] = lax.rsqrt(d0_ref[...] + d1_ref[...] + 1.0)


def _scale_split_kernel(x_ref, w_ref, dinv_ref, o0, o1, o2, o3):
    h = jnp.dot(x_ref[...], w_ref[...], preferred_element_type=jnp.float32)
    g = h * dinv_ref[...]
    o0[...] = g[:, 0:16]
    o1[...] = g[:, 16:32]
    o2[...] = g[:, 32:48]
    o3[...] = g[:, 48:64]


def _mid_kernel(a0, a1, a2, a3, dinv_ref, b1_ref, w2_ref, o_ref):
    agg = jnp.concatenate([a0[...], a1[...], a2[...], a3[...]], axis=1)
    h1 = jax.nn.relu(agg * dinv_ref[...] + b1_ref[...])
    g2 = jnp.dot(h1, w2_ref[...], preferred_element_type=jnp.float32)
    o_ref[...] = g2 * dinv_ref[...]


def _final_kernel(p0, p1, dinv_ref, b2_ref, o_ref):
    v = (p0[...] + p1[...]) * dinv_ref[...]
    o_ref[...] = jax.nn.sigmoid(v[:, 0:2] + b2_ref[...])


def kernel(x, edge_index, W1, b1, W2, b2):
    src = edge_index[0]
    dst = edge_index[1]

    ones1 = jnp.ones((_B,), jnp.float32)
    zeros1 = jnp.zeros((_B,), jnp.float32)
    zeros2 = jnp.zeros((_B, 16), jnp.float32)

    d0, d1 = _make_deg_kernel()(dst, ones1, zeros1)

    dinvp = pl.pallas_call(
        _dinv_kernel,
        out_shape=jax.ShapeDtypeStruct((782, 128), jnp.float32),
    )(d0.reshape(782, 128), d1.reshape(782, 128))
    dinv2d = dinvp.reshape(_NP, 1)

    W1p = jnp.pad(W1, ((0, 0), (0, 14)))
    grid = _N // _B
    tab_sds = jax.ShapeDtypeStruct((_NP, 16), jnp.float32)
    tab_spec = pl.BlockSpec((_B, 16), lambda i: (i, 0))
    g1c = pl.pallas_call(
        _scale_split_kernel,
        grid=(grid,),
        in_specs=[
            pl.BlockSpec((_B, 768), lambda i: (i, 0)),
            pl.BlockSpec((768, 64), lambda i: (0, 0)),
            pl.BlockSpec((_B, 1), lambda i: (i, 0)),
        ],
        out_specs=[tab_spec] * 4,
        out_shape=[tab_sds] * 4,
    )(x, W1p, dinv2d)

    prop_chunk = _make_prop(_E, init1_fixed=False)
    agg0, agg1 = prop_chunk(g1c[0], g1c[1], zeros2, src, dst)
    agg2, agg3 = prop_chunk(g1c[2], g1c[3], zeros2, src, dst)

    b1p = jnp.pad(b1, (0, 14)).reshape(1, 64)
    W2p = jnp.pad(W2, ((0, 14), (0, 14)))
    g2 = pl.pallas_call(
        _mid_kernel,
        grid=(grid,),
        in_specs=[tab_spec, tab_spec, tab_spec, tab_spec,
                  pl.BlockSpec((_B, 1), lambda i: (i, 0)),
                  pl.BlockSpec((1, 64), lambda i: (0, 0)),
                  pl.BlockSpec((64, 16), lambda i: (0, 0))],
        out_specs=tab_spec,
        out_shape=tab_sds,
    )(agg0, agg1, agg2, agg3, dinv2d, b1p, W2p)

    prop_half = _make_prop(_E // 2, init1_fixed=True)
    p0, p1 = prop_half(g2, g2, zeros2, src, dst)

    b2r = b2.reshape(1, 2)
    out = pl.pallas_call(
        _final_kernel,
        grid=(grid,),
        in_specs=[tab_spec, tab_spec,
                  pl.BlockSpec((_B, 1), lambda i: (i, 0)),
                  pl.BlockSpec((1, 2), lambda i: (0, 0))],
        out_specs=pl.BlockSpec((_B, 2), lambda i: (i, 0)),
        out_shape=jax.ShapeDtypeStruct((_N, 2), jnp.float32),
    )(p0, p1, dinv2d, b2r)
    return out
